# baseline (device time: 35268 ns/iter reference)
import jax
import jax.numpy as jnp
from jax import lax
from jax.experimental import pallas as pl
from jax.experimental.pallas import tpu as pltpu

N_DEV = 16
G = 8


def kernel(x):
    m, n = x.shape
    n_groups = m // G

    def body(x_ref, out_ref, msg_ref, recv_ref, send_sems, recv_sems):
        my = lax.axis_index("i")

        xv = x_ref[...]

        t = xv
        rows = m
        while rows > 1:
            half = rows // 2
            t = t[:half, :] * t[half : 2 * half, :]
            rows = half
        msg_ref[...] = t

        def pair_rdma(j):
            return pltpu.make_async_remote_copy(
                src_ref=msg_ref,
                dst_ref=recv_ref.at[j - 1],
                send_sem=send_sems.at[j - 1],
                recv_sem=recv_sems.at[j - 1],
                device_id=((my + j) % N_DEV,),
                device_id_type=pl.DeviceIdType.MESH,
            )

        for j in range(1, N_DEV):
            @pl.when(my + j < N_DEV)
            def _():
                pair_rdma(j).start()

        row_in_group = lax.broadcasted_iota(jnp.int32, (m, 1), 0) % G
        v = xv
        s = 1
        while s < G:
            shifted = jnp.concatenate(
                [jnp.ones((s, n), v.dtype), v[: m - s, :]], axis=0
            )
            v = jnp.where(row_in_group >= s, v * shifted, v)
            s *= 2

        wg = v.reshape(n_groups, G, n)
        gt = wg[:, G - 1, :]
        s = 1
        while s < n_groups:
            shifted = jnp.concatenate(
                [jnp.ones((s, n), gt.dtype), gt[: n_groups - s, :]], axis=0
            )
            gt = gt * shifted
            s *= 2
        excl_g = jnp.concatenate(
            [jnp.ones((1, n), gt.dtype), gt[: n_groups - 1, :]], axis=0
        )

        e_val = jnp.ones((1, n), jnp.float32)
        for j in range(1, N_DEV):
            @pl.when(my >= j)
            def _():
                pair_rdma(j).wait_recv()

            q = recv_ref[j - 1, :, :]
            e_val = jnp.where(my >= j, e_val * q, e_val)

        for j in range(1, N_DEV):
            @pl.when(my + j < N_DEV)
            def _():
                pair_rdma(j).wait_send()

        out_ref[...] = (
            wg * excl_g[:, None, :] * e_val[None, :, :]
        ).reshape(m, n)

    return pl.pallas_call(
        body,
        out_shape=jax.ShapeDtypeStruct((m, n), jnp.float32),
        in_specs=[pl.BlockSpec(memory_space=pltpu.VMEM)],
        out_specs=pl.BlockSpec(memory_space=pltpu.VMEM),
        scratch_shapes=[
            pltpu.VMEM((1, n), jnp.float32),
            pltpu.VMEM((N_DEV - 1, 1, n), jnp.float32),
            pltpu.SemaphoreType.DMA((N_DEV - 1,)),
            pltpu.SemaphoreType.DMA((N_DEV - 1,)),
        ],
        compiler_params=pltpu.CompilerParams(has_side_effects=True),
    )(x)


# device time: 18239 ns/iter; 1.9337x vs baseline; 1.9337x over previous
import jax
import jax.numpy as jnp
from jax import lax
from jax.experimental import pallas as pl
from jax.experimental.pallas import tpu as pltpu

N_DEV = 16
SEND_AFTER_STEP = 9


def kernel(x):
    m, n = x.shape

    def body(x_ref, out_ref, msg_ref, recv_ref, send_sems, recv_sems):
        my = lax.axis_index("i")

        barrier_sem = pltpu.get_barrier_semaphore()
        for j in range(1, N_DEV):
            pl.semaphore_signal(
                barrier_sem,
                inc=1,
                device_id=((my + j) % N_DEV,),
                device_id_type=pl.DeviceIdType.MESH,
            )

        xv = x_ref[...]

        t = xv
        rows = m
        while rows > 1:
            half = rows // 2
            t = t[:half, :] * t[half : 2 * half, :]
            rows = half
        msg_ref[...] = t

        shifts = []
        s = 1
        while s < m:
            shifts.append(s)
            s *= 2

        def hs_step(v, sh):
            shifted = jnp.concatenate(
                [jnp.ones((sh, n), v.dtype), v[: m - sh, :]], axis=0
            )
            return v * shifted

        v = xv
        for sh in shifts[:SEND_AFTER_STEP]:
            v = hs_step(v, sh)

        pl.semaphore_wait(barrier_sem, N_DEV - 1)

        def pair_rdma(d):
            return pltpu.make_async_remote_copy(
                src_ref=msg_ref,
                dst_ref=recv_ref.at[d - 1],
                send_sem=send_sems.at[d - 1],
                recv_sem=recv_sems.at[d - 1],
                device_id=((my + d) % N_DEV,),
                device_id_type=pl.DeviceIdType.MESH,
            )

        for d in range(1, N_DEV):
            @pl.when(my + d < N_DEV)
            def _():
                pair_rdma(d).start()

        for sh in shifts[SEND_AFTER_STEP:]:
            v = hs_step(v, sh)

        e_val = jnp.ones((1, n), jnp.float32)
        for d in range(1, N_DEV):
            @pl.when(my >= d)
            def _():
                pair_rdma(d).wait_recv()

            q = recv_ref[d - 1, :, :]
            e_val = jnp.where(my >= d, e_val * q, e_val)

        for d in range(1, N_DEV):
            @pl.when(my + d < N_DEV)
            def _():
                pair_rdma(d).wait_send()

        out_ref[...] = v * e_val

    return pl.pallas_call(
        body,
        out_shape=jax.ShapeDtypeStruct((m, n), jnp.float32),
        in_specs=[pl.BlockSpec(memory_space=pltpu.VMEM)],
        out_specs=pl.BlockSpec(memory_space=pltpu.VMEM),
        scratch_shapes=[
            pltpu.VMEM((1, n), jnp.float32),
            pltpu.VMEM((N_DEV - 1, 1, n), jnp.float32),
            pltpu.SemaphoreType.DMA((N_DEV - 1,)),
            pltpu.SemaphoreType.DMA((N_DEV - 1,)),
        ],
        compiler_params=pltpu.CompilerParams(
            has_side_effects=True, collective_id=0
        ),
    )(x)
